# baseline (speedup 1.0000x reference)
import functools

import jax
import jax.numpy as jnp
from jax import lax
from jax.experimental import pallas as pl
from jax.experimental.pallas import tpu as pltpu
from jax.experimental.pallas import tpu_sc as plsc

NUM_EMBEDDINGS = 1000000
EMBEDDING_DIM = 64
TOTAL = 16384 * 20

_info = plsc.get_sparse_core_info()
_NC, _NS = _info.num_cores, _info.num_subcores
_NW = _NC * _NS
_B_PER_W = TOTAL // _NW
_CH = 512
_N_CHUNKS = _B_PER_W // _CH


def _make_kernel():
    mesh = plsc.VectorSubcoreMesh(core_axis_name="c", subcore_axis_name="s")

    @functools.partial(
        pl.kernel,
        mesh=mesh,
        out_type=jax.ShapeDtypeStruct((TOTAL, EMBEDDING_DIM), jnp.float32),
        compiler_params=pltpu.CompilerParams(use_tc_tiling_on_sc=False),
        scratch_types=[
            pltpu.VMEM((_CH,), jnp.int32),
            pltpu.VMEM((_CH, EMBEDDING_DIM), jnp.float32),
            pltpu.SemaphoreType.DMA,
        ],
    )
    def gather_kernel(table_hbm, idx_hbm, out_hbm, idx_v, rows_v, sem):
        wid = lax.axis_index("s") * _NC + lax.axis_index("c")
        base = wid * _B_PER_W

        def body(i, carry):
            off = base + i * _CH
            pltpu.sync_copy(idx_hbm.at[pl.ds(off, _CH)], idx_v)
            pltpu.async_copy(table_hbm.at[idx_v], rows_v, sem).wait()
            pltpu.sync_copy(rows_v, out_hbm.at[pl.ds(off, _CH)])
            return carry

        lax.fori_loop(0, _N_CHUNKS, body, 0)

    return gather_kernel


_gather = _make_kernel()


def kernel(token_ids, embedding):
    flat_ids = token_ids.reshape(TOTAL)
    out = _gather(embedding, flat_ids)
    return out.reshape(token_ids.shape[0], token_ids.shape[1], EMBEDDING_DIM)


# speedup vs baseline: 1.0257x; 1.0257x over previous
import functools

import jax
import jax.numpy as jnp
from jax import lax
from jax.experimental import pallas as pl
from jax.experimental.pallas import tpu as pltpu
from jax.experimental.pallas import tpu_sc as plsc

NUM_EMBEDDINGS = 1000000
EMBEDDING_DIM = 64
TOTAL = 16384 * 20

_info = plsc.get_sparse_core_info()
_NC, _NS = _info.num_cores, _info.num_subcores
_NW = _NC * _NS
_B_PER_W = TOTAL // _NW
_CH = 640
_N_CHUNKS = _B_PER_W // _CH


def _make_kernel():
    mesh = plsc.VectorSubcoreMesh(core_axis_name="c", subcore_axis_name="s")

    @functools.partial(
        pl.kernel,
        mesh=mesh,
        out_type=jax.ShapeDtypeStruct((TOTAL, EMBEDDING_DIM), jnp.float32),
        compiler_params=pltpu.CompilerParams(use_tc_tiling_on_sc=False),
        scratch_types=[
            pltpu.VMEM((_B_PER_W,), jnp.int32),
            pltpu.VMEM((_CH, EMBEDDING_DIM), jnp.float32),
            pltpu.VMEM((_CH, EMBEDDING_DIM), jnp.float32),
            pltpu.SemaphoreType.DMA,
            pltpu.SemaphoreType.DMA,
            pltpu.SemaphoreType.DMA,
            pltpu.SemaphoreType.DMA,
        ],
    )
    def gather_kernel(table_hbm, idx_hbm, out_hbm, idx_v, rows0, rows1,
                      gs0, gs1, os0, os1):
        wid = lax.axis_index("s") * _NC + lax.axis_index("c")
        base = wid * _B_PER_W

        pltpu.sync_copy(idx_hbm.at[pl.ds(base, _B_PER_W)], idx_v)

        rows = (rows0, rows1)
        gsem = (gs0, gs1)
        osem = (os0, os1)
        gathers = [None, None]
        writes = [None, None]

        gathers[0] = pltpu.async_copy(
            table_hbm.at[idx_v.at[pl.ds(0, _CH)]], rows[0], gsem[0])
        for i in range(_N_CHUNKS):
            b = i % 2
            nb = (i + 1) % 2
            if i + 1 < _N_CHUNKS:
                if writes[nb] is not None:
                    writes[nb].wait()
                gathers[nb] = pltpu.async_copy(
                    table_hbm.at[idx_v.at[pl.ds((i + 1) * _CH, _CH)]],
                    rows[nb], gsem[nb])
            gathers[b].wait()
            writes[b] = pltpu.async_copy(
                rows[b], out_hbm.at[pl.ds(base + i * _CH, _CH)], osem[b])
        for b in range(2):
            if writes[b] is not None:
                writes[b].wait()

    return gather_kernel


_gather = _make_kernel()


def kernel(token_ids, embedding):
    flat_ids = token_ids.reshape(TOTAL)
    out = _gather(embedding, flat_ids)
    return out.reshape(token_ids.shape[0], token_ids.shape[1], EMBEDDING_DIM)


# speedup vs baseline: 1.0605x; 1.0340x over previous
import functools

import jax
import jax.numpy as jnp
from jax import lax
from jax.experimental import pallas as pl
from jax.experimental.pallas import tpu as pltpu
from jax.experimental.pallas import tpu_sc as plsc

NUM_EMBEDDINGS = 1000000
EMBEDDING_DIM = 64
PAD_DIM = 128
TOTAL = 16384 * 20

_info = plsc.get_sparse_core_info()
_NC, _NS = _info.num_cores, _info.num_subcores
_NW = _NC * _NS
_B_PER_W = TOTAL // _NW
_CH = 320
_N_CHUNKS = _B_PER_W // _CH


def _make_kernel():
    mesh = plsc.VectorSubcoreMesh(core_axis_name="c", subcore_axis_name="s")

    @functools.partial(
        pl.kernel,
        mesh=mesh,
        out_type=jax.ShapeDtypeStruct((TOTAL, EMBEDDING_DIM), jnp.float32),
        compiler_params=pltpu.CompilerParams(use_tc_tiling_on_sc=False),
        scratch_types=[
            pltpu.VMEM((_B_PER_W,), jnp.int32),
            pltpu.VMEM((_CH, PAD_DIM), jnp.float32),
            pltpu.VMEM((_CH, PAD_DIM), jnp.float32),
            pltpu.SemaphoreType.DMA,
            pltpu.SemaphoreType.DMA,
            pltpu.SemaphoreType.DMA,
            pltpu.SemaphoreType.DMA,
        ],
    )
    def gather_kernel(table_hbm, idx_hbm, out_hbm, idx_v, rows0, rows1,
                      gs0, gs1, os0, os1):
        wid = lax.axis_index("s") * _NC + lax.axis_index("c")
        base = wid * _B_PER_W

        pltpu.sync_copy(idx_hbm.at[pl.ds(base, _B_PER_W)], idx_v)

        rows = (rows0, rows1)
        gsem = (gs0, gs1)
        osem = (os0, os1)
        gathers = [None, None]
        writes = [None, None]

        gathers[0] = pltpu.async_copy(
            table_hbm.at[idx_v.at[pl.ds(0, _CH)]], rows[0], gsem[0])
        for i in range(_N_CHUNKS):
            b = i % 2
            nb = (i + 1) % 2
            if i + 1 < _N_CHUNKS:
                if writes[nb] is not None:
                    writes[nb].wait()
                gathers[nb] = pltpu.async_copy(
                    table_hbm.at[idx_v.at[pl.ds((i + 1) * _CH, _CH)]],
                    rows[nb], gsem[nb])
            gathers[b].wait()
            writes[b] = pltpu.async_copy(
                rows[b].at[:, pl.ds(0, EMBEDDING_DIM)],
                out_hbm.at[pl.ds(base + i * _CH, _CH)], osem[b])
        for b in range(2):
            if writes[b] is not None:
                writes[b].wait()

    return gather_kernel


_gather = _make_kernel()


def kernel(token_ids, embedding):
    emb_pad = jnp.pad(embedding, ((0, 0), (0, PAD_DIM - EMBEDDING_DIM)))
    flat_ids = token_ids.reshape(TOTAL)
    out = _gather(emb_pad, flat_ids)
    return out.reshape(token_ids.shape[0], token_ids.shape[1], EMBEDDING_DIM)
